# baseline (speedup 1.0000x reference)
import jax
import jax.numpy as jnp
from jax import lax
from jax.experimental import pallas as pl
from jax.experimental.pallas import tpu as pltpu


_ROWS = 16384
_COLS = 4096
_RB = 128
_NBUF = 8
_NCH = _ROWS // _RB


def _body(in_ref, out_ref, buf, sin, sout):
    def _in(c, b):
        return pltpu.make_async_copy(
            in_ref.at[pl.ds(c * _RB, _RB), :], buf.at[b], sin.at[b])

    def _out(c, b):
        return pltpu.make_async_copy(
            buf.at[b], out_ref.at[pl.ds(c * _RB, _RB), :], sout.at[b])

    for b in range(_NBUF):
        _in(b, b).start()

    def step(it, carry):
        for b in range(_NBUF):
            c = it * _NBUF + b
            _in(c, b).wait()
            _out(c, b).start()
            _out(c, b).wait()
            nc = c + _NBUF

            @pl.when(nc < _NCH)
            def _():
                _in(nc, b).start()
        return carry

    lax.fori_loop(0, _NCH // _NBUF, step, 0)


def kernel(tensor):
    flat = tensor.reshape(_ROWS, _COLS)
    out = pl.pallas_call(
        _body,
        in_specs=[pl.BlockSpec(memory_space=pl.ANY)],
        out_specs=pl.BlockSpec(memory_space=pl.ANY),
        out_shape=jax.ShapeDtypeStruct((_ROWS, _COLS), jnp.float32),
        scratch_shapes=[
            pltpu.VMEM((_NBUF, _RB, _COLS), jnp.float32),
            pltpu.SemaphoreType.DMA((_NBUF,)),
            pltpu.SemaphoreType.DMA((_NBUF,)),
        ],
    )(flat)
    return out.reshape(tensor.shape[0], tensor.shape[1], _COLS)


# speedup vs baseline: 1.0436x; 1.0436x over previous
import jax
import jax.numpy as jnp
from jax import lax
from jax.experimental import pallas as pl
from jax.experimental.pallas import tpu as pltpu


_ROWS = 16384
_COLS = 4096
_RB = 128
_NBUF = 8
_NCH = _ROWS // _RB


def _body(in_ref, out_ref, buf, sin, sout):
    def _in(c, b):
        return pltpu.make_async_copy(
            in_ref.at[pl.ds(c * _RB, _RB), :], buf.at[b], sin.at[b])

    def _out(c, b):
        return pltpu.make_async_copy(
            buf.at[b], out_ref.at[pl.ds(c * _RB, _RB), :], sout.at[b])

    for b in range(_NBUF):
        _in(b, b).start()

    def step(it, carry):
        base = it * _NBUF
        for b in range(_NBUF):
            _in(base + b, b).wait()
            _out(base + b, b).start()
        for b in range(_NBUF):
            nc = base + b + _NBUF

            @pl.when(nc < _NCH)
            def _():
                _out(base + b, b).wait()
                _in(nc, b).start()
        return carry

    lax.fori_loop(0, _NCH // _NBUF, step, 0)
    for b in range(_NBUF):
        _out(_NCH - _NBUF + b, b).wait()


def kernel(tensor):
    flat = tensor.reshape(_ROWS, _COLS)
    out = pl.pallas_call(
        _body,
        in_specs=[pl.BlockSpec(memory_space=pl.ANY)],
        out_specs=pl.BlockSpec(memory_space=pl.ANY),
        out_shape=jax.ShapeDtypeStruct((_ROWS, _COLS), jnp.float32),
        scratch_shapes=[
            pltpu.VMEM((_NBUF, _RB, _COLS), jnp.float32),
            pltpu.SemaphoreType.DMA((_NBUF,)),
            pltpu.SemaphoreType.DMA((_NBUF,)),
        ],
    )(flat)
    return out.reshape(tensor.shape[0], tensor.shape[1], _COLS)
